# baseline (speedup 1.0000x reference)
import functools

import jax
import jax.numpy as jnp
from jax.experimental import pallas as pl
from jax.experimental.pallas import tpu as pltpu

_SIZES = (128, 256, 512, 768, 1024, 768, 384, 256)


def _lora_pipe(x_hbm, tid_ref, a_hbm, b_ref, out_hbm, xbufs, obufs, abuf,
               a_flat, in_sems, out_sems, a_sem, *, rank):
    n_tasks = abuf.shape[0]
    n_cols = n_tasks * rank
    offs = []
    off = 0
    for sz in _SIZES:
        offs.append(off)
        off += sz

    pltpu.make_async_copy(a_hbm, abuf, a_sem).start()
    for k, (o, sz) in enumerate(zip(offs, _SIZES)):
        pltpu.make_async_copy(
            x_hbm.at[pl.ds(o, sz)], xbufs[k], in_sems.at[k]).start()

    pltpu.make_async_copy(a_hbm, abuf, a_sem).wait()
    a_flat[...] = jnp.concatenate([abuf[e] for e in range(n_tasks)], axis=1)

    for k, (o, sz) in enumerate(zip(offs, _SIZES)):
        pltpu.make_async_copy(
            x_hbm.at[pl.ds(o, sz)], xbufs[k], in_sems.at[k]).wait()
        tid = jnp.reshape(tid_ref[pl.ds(o, sz)], (sz, 1))
        col_expert = jax.lax.broadcasted_iota(jnp.int32, (sz, n_cols), 1) // rank
        xa = jnp.dot(xbufs[k][...], a_flat[...], preferred_element_type=jnp.float32)
        xa = jnp.where(tid == col_expert, xa, 0.0)
        obufs[k][...] = jnp.dot(xa, b_ref[...], preferred_element_type=jnp.float32)
        pltpu.make_async_copy(
            obufs[k], out_hbm.at[pl.ds(o, sz)], out_sems.at[k]).start()

    for k, (o, sz) in enumerate(zip(offs, _SIZES)):
        pltpu.make_async_copy(
            obufs[k], out_hbm.at[pl.ds(o, sz)], out_sems.at[k]).wait()


def kernel(x, task_id, lora_A, lora_B):
    T, in_dim = x.shape
    n_tasks, _, rank = lora_A.shape
    out_dim = lora_B.shape[2]
    er = n_tasks * rank
    assert sum(_SIZES) == T

    b_flat = lora_B.reshape(er, out_dim)

    nc = len(_SIZES)
    body = functools.partial(_lora_pipe, rank=rank)
    return pl.pallas_call(
        body,
        in_specs=[
            pl.BlockSpec(memory_space=pl.ANY),
            pl.BlockSpec((T,), lambda: (0,)),
            pl.BlockSpec(memory_space=pl.ANY),
            pl.BlockSpec((er, out_dim), lambda: (0, 0)),
        ],
        out_specs=pl.BlockSpec(memory_space=pl.ANY),
        out_shape=jax.ShapeDtypeStruct((T, out_dim), jnp.float32),
        scratch_shapes=(
            [pltpu.VMEM((sz, in_dim), jnp.float32) for sz in _SIZES],
            [pltpu.VMEM((sz, out_dim), jnp.float32) for sz in _SIZES],
            pltpu.VMEM((n_tasks, in_dim, rank), jnp.float32),
            pltpu.VMEM((in_dim, er), jnp.float32),
            pltpu.SemaphoreType.DMA((nc,)),
            pltpu.SemaphoreType.DMA((nc,)),
            pltpu.SemaphoreType.DMA,
        ),
    )(x, task_id, lora_A, b_flat)


# speedup vs baseline: 1.7449x; 1.7449x over previous
import functools

import jax
import jax.numpy as jnp
from jax.experimental import pallas as pl
from jax.experimental.pallas import tpu as pltpu

_SIZES = (128, 256, 512, 768, 1024, 768, 384, 256)


def _lora_pipe(x_hbm, tid_ref, a_ref, b_ref, out_hbm, xbufs, obufs, in_sems,
               out_sems, *, rank):
    n_cols = a_ref.shape[1]
    offs = []
    off = 0
    for sz in _SIZES:
        offs.append(off)
        off += sz

    for k, (o, sz) in enumerate(zip(offs, _SIZES)):
        pltpu.make_async_copy(
            x_hbm.at[pl.ds(o, sz)], xbufs[k], in_sems.at[k]).start()

    for k, (o, sz) in enumerate(zip(offs, _SIZES)):
        pltpu.make_async_copy(
            x_hbm.at[pl.ds(o, sz)], xbufs[k], in_sems.at[k]).wait()
        tid = jnp.reshape(tid_ref[pl.ds(o, sz)], (sz, 1))
        col_expert = jax.lax.broadcasted_iota(jnp.int32, (sz, n_cols), 1) // rank
        xa = jnp.dot(xbufs[k][...], a_ref[...], preferred_element_type=jnp.float32)
        xa = jnp.where(tid == col_expert, xa, 0.0)
        obufs[k][...] = jnp.dot(xa, b_ref[...], preferred_element_type=jnp.float32)
        pltpu.make_async_copy(
            obufs[k], out_hbm.at[pl.ds(o, sz)], out_sems.at[k]).start()

    for k, (o, sz) in enumerate(zip(offs, _SIZES)):
        pltpu.make_async_copy(
            obufs[k], out_hbm.at[pl.ds(o, sz)], out_sems.at[k]).wait()


def kernel(x, task_id, lora_A, lora_B):
    T, in_dim = x.shape
    n_tasks, _, rank = lora_A.shape
    out_dim = lora_B.shape[2]
    er = n_tasks * rank
    assert sum(_SIZES) == T

    a_flat = jnp.transpose(lora_A, (1, 0, 2)).reshape(in_dim, er)
    b_flat = lora_B.reshape(er, out_dim)

    nc = len(_SIZES)
    body = functools.partial(_lora_pipe, rank=rank)
    return pl.pallas_call(
        body,
        in_specs=[
            pl.BlockSpec(memory_space=pl.ANY),
            pl.BlockSpec((T,), lambda: (0,)),
            pl.BlockSpec((in_dim, er), lambda: (0, 0)),
            pl.BlockSpec((er, out_dim), lambda: (0, 0)),
        ],
        out_specs=pl.BlockSpec(memory_space=pl.ANY),
        out_shape=jax.ShapeDtypeStruct((T, out_dim), jnp.float32),
        scratch_shapes=(
            [pltpu.VMEM((sz, in_dim), jnp.float32) for sz in _SIZES],
            [pltpu.VMEM((sz, out_dim), jnp.float32) for sz in _SIZES],
            pltpu.SemaphoreType.DMA((nc,)),
            pltpu.SemaphoreType.DMA((nc,)),
        ),
    )(x, task_id, a_flat, b_flat)
